# Initial kernel scaffold; baseline (speedup 1.0000x reference)
import jax
import jax.numpy as jnp
from jax.experimental import pallas as pl


def kernel(x, edge_index, W1, b1, W2, b2, W3, b3):
    raise NotImplementedError("write your pallas kernel here")


# speedup vs baseline: 18.5240x; 18.5240x over previous
import functools

import jax
import jax.numpy as jnp
from jax import lax
from jax.experimental import pallas as pl
from jax.experimental.pallas import tpu as pltpu
from jax.experimental.pallas import tpu_sc as plsc

N_NODES = 10000
N_EDGES = 320000
D = 128

NC = 2
NS = 16
NW = NC * NS
E_W = N_EDGES // NW
C = 200
K = E_W // C
NP = 10240
RPT = NP // NS

_mesh = plsc.VectorSubcoreMesh(core_axis_name="c", subcore_axis_name="s")


def _zero_fill(buf, rows, width):
    for r in range(rows):
        for c in range(width // 16):
            buf.at[pl.ds(r, 1), pl.ds(c * 16, 16)][...] = jnp.zeros(
                (1, 16), jnp.float32)


@functools.partial(
    pl.kernel,
    out_type=jax.ShapeDtypeStruct((NC, NP, 16), jnp.float32),
    mesh=_mesh,
    scratch_types=[
        pltpu.VMEM((C,), jnp.int32),
        pltpu.VMEM((C, 16), jnp.float32),
        pltpu.VMEM((8, 16), jnp.float32),
        pltpu.VMEM_SHARED((NP, 16), jnp.float32),
    ],
)
def _sc_hist(dst_hbm, out_hbm, dst_v, ones_v, zbuf, table):
    cid = lax.axis_index("c")
    sid = lax.axis_index("s")
    wid = sid * NC + cid

    _zero_fill(zbuf, 8, 16)

    @pl.loop(0, C)
    def _(i):
        ones_v.at[pl.ds(i, 1), pl.ds(0, 16)][...] = jnp.ones((1, 16),
                                                             jnp.float32)

    base_r = sid * RPT

    @pl.loop(0, RPT // 8)
    def _(i):
        pltpu.sync_copy(zbuf, table.at[pl.ds(base_r + i * 8, 8)])

    plsc.subcore_barrier()

    base_e = wid * E_W

    @pl.loop(0, K)
    def _(j):
        pltpu.sync_copy(dst_hbm.at[pl.ds(base_e + j * C, C)], dst_v)
        pltpu.sync_copy(ones_v, table.at[dst_v], add=True)

    plsc.subcore_barrier()
    pltpu.sync_copy(table.at[pl.ds(base_r, RPT)],
                    out_hbm.at[cid, pl.ds(base_r, RPT)])


@functools.partial(
    pl.kernel,
    out_type=jax.ShapeDtypeStruct((NC, NP, D), jnp.float32),
    mesh=_mesh,
    scratch_types=[
        pltpu.VMEM((C,), jnp.int32),
        pltpu.VMEM((C,), jnp.int32),
        pltpu.VMEM((C, D), jnp.float32),
        pltpu.VMEM((8, D), jnp.float32),
        pltpu.VMEM_SHARED((NP, D), jnp.float32),
        pltpu.SemaphoreType.DMA,
    ],
)
def _sc_scatter(hs_hbm, src_hbm, dst_hbm, out_hbm, src_v, dst_v, rows_v,
                zbuf, table, sem):
    cid = lax.axis_index("c")
    sid = lax.axis_index("s")
    wid = sid * NC + cid

    _zero_fill(zbuf, 8, D)
    base_r = sid * RPT

    @pl.loop(0, RPT // 8)
    def _(i):
        pltpu.sync_copy(zbuf, table.at[pl.ds(base_r + i * 8, 8)])

    plsc.subcore_barrier()

    base_e = wid * E_W

    @pl.loop(0, K)
    def _(j):
        pltpu.sync_copy(src_hbm.at[pl.ds(base_e + j * C, C)], src_v)
        pltpu.sync_copy(dst_hbm.at[pl.ds(base_e + j * C, C)], dst_v)
        pltpu.async_copy(hs_hbm.at[src_v], rows_v, sem).wait()
        pltpu.sync_copy(rows_v, table.at[dst_v], add=True)

    plsc.subcore_barrier()
    pltpu.sync_copy(table.at[pl.ds(base_r, RPT)],
                    out_hbm.at[cid, pl.ds(base_r, RPT)])


BLK = 2000
_GRID = N_NODES // BLK


def _row_spec(w):
    return pl.BlockSpec((BLK, w), lambda i: (i, 0))


def _full_spec(a, b):
    return pl.BlockSpec((a, b), lambda i: (0, 0))


def _mm_body(x_ref, w_ref, o_ref):
    o_ref[...] = jnp.dot(x_ref[...], w_ref[...],
                         preferred_element_type=jnp.float32)


def _mm(x, w):
    return pl.pallas_call(
        _mm_body,
        grid=(_GRID,),
        in_specs=[_row_spec(D), _full_spec(D, D)],
        out_specs=_row_spec(D),
        out_shape=jax.ShapeDtypeStruct((N_NODES, D), jnp.float32),
    )(x, w)


def _dinv(d0_ref, d1_ref):
    deg = d0_ref[...][:, :1] + d1_ref[...][:, :1] + 1.0
    return lax.rsqrt(deg)


def _scale_body(h_ref, d0_ref, d1_ref, o_ref):
    o_ref[...] = h_ref[...] * _dinv(d0_ref, d1_ref)


def _scale(h, d0, d1):
    return pl.pallas_call(
        _scale_body,
        grid=(_GRID,),
        in_specs=[_row_spec(D), _row_spec(16), _row_spec(16)],
        out_specs=_row_spec(D),
        out_shape=jax.ShapeDtypeStruct((N_NODES, D), jnp.float32),
    )(h, d0, d1)


def _mid_body(p0_ref, p1_ref, hs_ref, d0_ref, d1_ref, b_ref, w_ref, o_ref):
    dinv = _dinv(d0_ref, d1_ref)
    h = jnp.maximum(
        dinv * (p0_ref[...] + p1_ref[...] + hs_ref[...]) + b_ref[...], 0.0)
    o_ref[...] = jnp.dot(h, w_ref[...],
                         preferred_element_type=jnp.float32) * dinv


def _mid(p0, p1, hs, d0, d1, b, w):
    return pl.pallas_call(
        _mid_body,
        grid=(_GRID,),
        in_specs=[_row_spec(D), _row_spec(D), _row_spec(D), _row_spec(16),
                  _row_spec(16), _full_spec(1, D), _full_spec(D, D)],
        out_specs=_row_spec(D),
        out_shape=jax.ShapeDtypeStruct((N_NODES, D), jnp.float32),
    )(p0, p1, hs, d0, d1, b, w)


def _out_body(p0_ref, p1_ref, hs_ref, d0_ref, d1_ref, b_ref, w_ref, b3_ref,
              o_ref):
    dinv = _dinv(d0_ref, d1_ref)
    h = jnp.maximum(
        dinv * (p0_ref[...] + p1_ref[...] + hs_ref[...]) + b_ref[...], 0.0)
    o_ref[...] = jnp.dot(h, w_ref[...],
                         preferred_element_type=jnp.float32) + b3_ref[...]


def _out(p0, p1, hs, d0, d1, b, w3p, b3p):
    return pl.pallas_call(
        _out_body,
        grid=(_GRID,),
        in_specs=[_row_spec(D), _row_spec(D), _row_spec(D), _row_spec(16),
                  _row_spec(16), _full_spec(1, D), _full_spec(D, D),
                  _full_spec(1, D)],
        out_specs=_row_spec(D),
        out_shape=jax.ShapeDtypeStruct((N_NODES, D), jnp.float32),
    )(p0, p1, hs, d0, d1, b, w3p, b3p)


def kernel(x, edge_index, W1, b1, W2, b2, W3, b3):
    src = edge_index[0].astype(jnp.int32)
    dst = edge_index[1].astype(jnp.int32)

    out_ch = W3.shape[1]
    w3p = jnp.zeros((D, D), jnp.float32).at[:, :out_ch].set(W3)
    b3p = jnp.zeros((1, D), jnp.float32).at[:, :out_ch].set(b3)

    deg_p = _sc_hist(dst)
    h_raw1 = _mm(x, W1)
    d0 = deg_p[0, :N_NODES]
    d1 = deg_p[1, :N_NODES]

    h1s = _scale(h_raw1, d0, d1)
    p1 = _sc_scatter(h1s, src, dst)
    h2s = _mid(p1[0, :N_NODES], p1[1, :N_NODES], h1s, d0, d1,
               b1.reshape(1, D), W2)
    p2 = _sc_scatter(h2s, src, dst)
    outp = _out(p2[0, :N_NODES], p2[1, :N_NODES], h2s, d0, d1,
                b2.reshape(1, D), w3p, b3p)
    return outp[:, :out_ch]
